# baseline (speedup 1.0000x reference)
import jax
import jax.numpy as jnp
from jax import lax
from jax.experimental import pallas as pl
from jax.experimental.pallas import tpu as pltpu
from jax.experimental.pallas import tpu_sc as plsc

NUM_PROTEINS = 1000000
LATENT_DIM = 32
BATCH = 16384

NC, NS, L = 2, 16, 16
NW = NC * NS
B_PER_W = BATCH // NW
IDX_COLS = 128
IDX_ROWS = B_PER_W // IDX_COLS
PC = IDX_COLS // L
EMB_PACK = 128 // LATENT_DIM
EMB_ROWS = NUM_PROTEINS // EMB_PACK
RE_ROWS = -(-NUM_PROTEINS // 128)


def _sc_body(p1_hbm, p2_hbm, emb_hbm, re_hbm, beta_hbm, out_hbm,
             idx1_v, idx2_v, rowz1_v, rowz2_v, rowr1_v, rowr2_v,
             subz1_v, subz2_v, subr1_v, subr2_v,
             z1p, z2p, rr1p, rr2p, beta_v, out_v, sem):
    wid = lax.axis_index("s") * NC + lax.axis_index("c")

    pltpu.sync_copy(p1_hbm.at[pl.ds(wid * IDX_ROWS, IDX_ROWS)], idx1_v)
    pltpu.sync_copy(p2_hbm.at[pl.ds(wid * IDX_ROWS, IDX_ROWS)], idx2_v)
    pltpu.sync_copy(beta_hbm, beta_v)

    for k in range(IDX_ROWS):
        for o in range(0, IDX_COLS, L):
            s = pl.ds(o, L)
            d = pl.ds(k * IDX_COLS + o, L)
            v1 = idx1_v[k, s]
            v2 = idx2_v[k, s]
            rowz1_v[k, s] = lax.shift_right_logical(v1, 2)
            rowz2_v[k, s] = lax.shift_right_logical(v2, 2)
            rowr1_v[k, s] = lax.shift_right_logical(v1, 7)
            rowr2_v[k, s] = lax.shift_right_logical(v2, 7)
            subz1_v[d] = lax.bitwise_and(v1, jnp.int32(EMB_PACK - 1)) * LATENT_DIM
            subz2_v[d] = lax.bitwise_and(v2, jnp.int32(EMB_PACK - 1)) * LATENT_DIM
            subr1_v[d] = lax.bitwise_and(v1, jnp.int32(127))
            subr2_v[d] = lax.bitwise_and(v2, jnp.int32(127))

    beta = beta_v[...]
    lane = lax.iota(jnp.int32, L)

    for p in range(IDX_ROWS):
        cps = [
            pltpu.async_copy(emb_hbm.at[rowz1_v.at[p]], z1p, sem),
            pltpu.async_copy(emb_hbm.at[rowz2_v.at[p]], z2p, sem),
            pltpu.async_copy(re_hbm.at[rowr1_v.at[p]], rr1p, sem),
            pltpu.async_copy(re_hbm.at[rowr2_v.at[p]], rr2p, sem),
        ]
        for cp in cps:
            cp.wait()

        def chunk(c, _):
            rows = c * L + lane
            g = pl.ds(p * IDX_COLS + c * L, L)
            sz1 = subz1_v[g]
            sz2 = subz2_v[g]
            acc = jnp.zeros((L,), jnp.float32)
            for j in range(LATENT_DIM):
                a = plsc.load_gather(z1p, [rows, sz1 + j])
                b = plsc.load_gather(z2p, [rows, sz2 + j])
                d = a - b
                acc = acc + d * d
            r1 = plsc.load_gather(rr1p, [rows, subr1_v[g]])
            r2 = plsc.load_gather(rr2p, [rows, subr2_v[g]])
            s = jnp.maximum(acc, jnp.float32(1e-35))
            i = lax.bitcast_convert_type(s, jnp.int32)
            i = jnp.int32(0x5F3759DF) - lax.shift_right_arithmetic(i, 1)
            y = lax.bitcast_convert_type(i, jnp.float32)
            for _ in range(3):
                y = y * (jnp.float32(1.5) - jnp.float32(0.5) * s * y * y)
            dist = s * y
            out_v[g] = r1 + r2 - beta * dist
            return ()

        lax.fori_loop(0, PC, chunk, ())

    pltpu.sync_copy(out_v, out_hbm.at[pl.ds(wid * B_PER_W, B_PER_W)])


@jax.jit
def _run(p1, p2, emb, re_tab, beta_vec):
    mesh = plsc.VectorSubcoreMesh(core_axis_name="c", subcore_axis_name="s",
                                  num_cores=NC, num_subcores=NS)
    return pl.kernel(
        _sc_body,
        out_type=jax.ShapeDtypeStruct((BATCH,), jnp.float32),
        mesh=mesh,
        compiler_params=pltpu.CompilerParams(needs_layout_passes=False),
        scratch_types=[
            pltpu.VMEM((IDX_ROWS, IDX_COLS), jnp.int32),
            pltpu.VMEM((IDX_ROWS, IDX_COLS), jnp.int32),
            pltpu.VMEM((IDX_ROWS, IDX_COLS), jnp.int32),
            pltpu.VMEM((IDX_ROWS, IDX_COLS), jnp.int32),
            pltpu.VMEM((IDX_ROWS, IDX_COLS), jnp.int32),
            pltpu.VMEM((IDX_ROWS, IDX_COLS), jnp.int32),
            pltpu.VMEM((B_PER_W,), jnp.int32),
            pltpu.VMEM((B_PER_W,), jnp.int32),
            pltpu.VMEM((B_PER_W,), jnp.int32),
            pltpu.VMEM((B_PER_W,), jnp.int32),
            pltpu.VMEM((IDX_COLS, 128), jnp.float32),
            pltpu.VMEM((IDX_COLS, 128), jnp.float32),
            pltpu.VMEM((IDX_COLS, 128), jnp.float32),
            pltpu.VMEM((IDX_COLS, 128), jnp.float32),
            pltpu.VMEM((L,), jnp.float32),
            pltpu.VMEM((B_PER_W,), jnp.float32),
            pltpu.SemaphoreType.DMA,
        ],
    )(p1, p2, emb, re_tab, beta_vec)


def kernel(protein1_idx, protein2_idx, isoform_embeddings, random_effects, beta_iso):
    beta_vec = jnp.full((L,), beta_iso, jnp.float32)
    p1 = protein1_idx.astype(jnp.int32).reshape(BATCH // IDX_COLS, IDX_COLS)
    p2 = protein2_idx.astype(jnp.int32).reshape(BATCH // IDX_COLS, IDX_COLS)
    emb = isoform_embeddings.reshape(EMB_ROWS, 128)
    re_pad = jnp.pad(random_effects[:, 0],
                     (0, RE_ROWS * 128 - NUM_PROTEINS)).reshape(RE_ROWS, 128)
    return _run(p1, p2, emb, re_pad, beta_vec)
